# baseline (speedup 1.0000x reference)
import functools

import jax
import jax.numpy as jnp
from jax import lax
from jax.experimental import pallas as pl
from jax.experimental.pallas import tpu as pltpu
from jax.experimental.pallas import tpu_sc as plsc

SEQ = 200
EMBED = 128
BATCH = 1024
NW = 32
ROWS_PER_W = BATCH * SEQ // NW
CH = 128
NCH = ROWS_PER_W // CH
LANES = 16
NBUF = 5


def _sc_body(idx_hbm, table_hbm, pos_hbm, out_hbm, idx_v, pos_v,
             r0, r1, r2, r3, r4, g0, g1, g2, g3, g4, o0, o1, o2, o3, o4):
    rows = [r0, r1, r2, r3, r4]
    gsem = [g0, g1, g2, g3, g4]
    osem = [o0, o1, o2, o3, o4]
    wid = lax.axis_index("s") * 2 + lax.axis_index("c")
    base_row = wid * ROWS_PER_W
    pltpu.sync_copy(idx_hbm.at[wid], idx_v)
    pltpu.sync_copy(pos_hbm, pos_v)

    def start_gather(j, b):
        return pltpu.async_copy(table_hbm.at[idx_v.at[j]], rows[b], gsem[b])

    def add_pos(b, pbase):
        def add_body(r, c):
            p = pbase + r
            p = jnp.where(p >= SEQ, p - SEQ, p)
            for j in range(EMBED // LANES):
                sl = pl.ds(j * LANES, LANES)
                plsc.addupdate(rows[b].at[r, sl], pos_v[p, sl])
            return c
        lax.fori_loop(0, CH, add_body, 0)

    pend_g = {j: start_gather(j, j % NBUF) for j in range(3)}
    pend_o = {}
    for j in range(NCH):
        b = j % NBUF
        pend_g.pop(j).wait()
        add_pos(b, (j * CH) % SEQ)
        pend_o[j] = pltpu.async_copy(
            rows[b], out_hbm.at[pl.ds(base_row + j * CH, CH)], osem[b])
        if j - 2 in pend_o:
            pend_o.pop(j - 2).wait()
        if j + 3 < NCH:
            pend_g[j + 3] = start_gather(j + 3, (j + 3) % NBUF)
    for j in sorted(pend_o):
        pend_o.pop(j).wait()


@jax.jit
def _run(idx3, token_table, position_table):
    mesh = plsc.VectorSubcoreMesh(core_axis_name="c", subcore_axis_name="s")
    fn = functools.partial(
        pl.kernel,
        out_type=jax.ShapeDtypeStruct((BATCH * SEQ, EMBED), jnp.float32),
        mesh=mesh,
        scratch_types=(
            [pltpu.VMEM((NCH, CH), jnp.int32),
             pltpu.VMEM((SEQ, EMBED), jnp.float32)]
            + [pltpu.VMEM((CH, EMBED), jnp.float32)] * NBUF
            + [pltpu.SemaphoreType.DMA] * (2 * NBUF)
        ),
    )(_sc_body)
    return fn(idx3, token_table, position_table)


def kernel(inputs, token_table, position_table):
    idx3 = inputs.astype(jnp.int32).reshape(NW, NCH, CH)
    out = _run(idx3, token_table, position_table)
    return out.reshape(BATCH, SEQ, EMBED)


# speedup vs baseline: 1.5818x; 1.5818x over previous
import functools

import jax
import jax.numpy as jnp
from jax import lax
from jax.experimental import pallas as pl
from jax.experimental.pallas import tpu as pltpu
from jax.experimental.pallas import tpu_sc as plsc

SEQ = 200
EMBED = 128
BATCH = 1024
NW = 32
SEQ_PER_W = BATCH // NW
HALF = SEQ // 2
LANES = 16
NBUF = 4


def _sc_body(idx_hbm, table_hbm, pos_hbm, out_hbm, pos_v,
             r0, r1, r2, r3, i0, i1, i2, i3,
             g0, g1, g2, g3, o0, o1, o2, o3, s0, s1, s2, s3):
    rows = [r0, r1, r2, r3]
    idxs = [i0, i1, i2, i3]
    gsem = [g0, g1, g2, g3]
    osem = [o0, o1, o2, o3]
    isem = [s0, s1, s2, s3]
    wid = lax.axis_index("s") * 2 + lax.axis_index("c")
    pltpu.sync_copy(pos_hbm, pos_v)

    def stage_idx(s):
        k = s % NBUF
        return pltpu.async_copy(idx_hbm.at[wid, s], idxs[k], isem[k])

    def start_gather(s, b):
        k = s % NBUF
        cp0 = pltpu.async_copy(
            table_hbm.at[idxs[k].at[0]], rows[b].at[pl.ds(0, HALF)], gsem[b])
        cp1 = pltpu.async_copy(
            table_hbm.at[idxs[k].at[1]], rows[b].at[pl.ds(HALF, HALF)], gsem[b])
        return (cp0, cp1)

    def add_pos(b):
        def add_body(r, c):
            for j in range(EMBED // LANES):
                sl = pl.ds(j * LANES, LANES)
                plsc.addupdate(rows[b].at[r, sl], pos_v[r, sl])
            return c
        lax.fori_loop(0, SEQ, add_body, 0)

    pend_i = {s: stage_idx(s) for s in range(3)}
    pend_g = {}
    for s in range(2):
        pend_i.pop(s).wait()
        pend_g[s] = start_gather(s, s)
    pend_o = {}
    for s in range(SEQ_PER_W):
        b = s % NBUF
        for cp in pend_g.pop(s):
            cp.wait()
        add_pos(b)
        pend_o[s] = pltpu.async_copy(
            rows[b], out_hbm.at[wid * SEQ_PER_W + s], osem[b])
        if s - 2 in pend_o:
            pend_o.pop(s - 2).wait()
        if s + 3 < SEQ_PER_W:
            pend_i[s + 3] = stage_idx(s + 3)
        if s + 2 < SEQ_PER_W:
            pend_i.pop(s + 2).wait()
            pend_g[s + 2] = start_gather(s + 2, (s + 2) % NBUF)
    for s in sorted(pend_o):
        pend_o.pop(s).wait()


@jax.jit
def _run(idx4, token_table, position_table):
    mesh = plsc.VectorSubcoreMesh(core_axis_name="c", subcore_axis_name="s")
    fn = functools.partial(
        pl.kernel,
        out_type=jax.ShapeDtypeStruct((BATCH, SEQ, EMBED), jnp.float32),
        mesh=mesh,
        scratch_types=(
            [pltpu.VMEM((SEQ, EMBED), jnp.float32)]
            + [pltpu.VMEM((SEQ, EMBED), jnp.float32)] * NBUF
            + [pltpu.VMEM((2, HALF), jnp.int32)] * NBUF
            + [pltpu.SemaphoreType.DMA] * (3 * NBUF)
        ),
    )(_sc_body)
    return fn(idx4, token_table, position_table)


def kernel(inputs, token_table, position_table):
    idx4 = inputs.astype(jnp.int32).reshape(NW, SEQ_PER_W, 2, HALF)
    return _run(idx4, token_table, position_table)
